# baseline (speedup 1.0000x reference)
import functools

import jax
import jax.numpy as jnp
from jax import lax
from jax.experimental import pallas as pl
from jax.experimental.pallas import tpu as pltpu
from jax.experimental.pallas import tpu_sc as plsc

NUM_KP = 10
H = 128
W = 128
C = 32
BS = 4

R = BS * (NUM_KP + 1) * H * W
NGRP = 8
CG = C // NGRP
WPG = 4
RW = R // WPG
CB = 1024
NCHUNK = RW // CB
L = 16
TS = H * W * CG


def _sc_deform(tab_g, gx, gy):
    mesh = plsc.VectorSubcoreMesh(
        core_axis_name="c", subcore_axis_name="s", num_cores=2, num_subcores=16
    )

    @functools.partial(
        pl.kernel,
        out_type=jax.ShapeDtypeStruct((R, NGRP, CG), jnp.float32),
        mesh=mesh,
        compiler_params=pltpu.CompilerParams(
            needs_layout_passes=False, use_tc_tiling_on_sc=False
        ),
        scratch_types=[
            pltpu.VMEM((TS,), jnp.float32),
            pltpu.VMEM((CB,), jnp.float32),
            pltpu.VMEM((CB,), jnp.float32),
            pltpu.VMEM((CB,), jnp.float32),
            pltpu.VMEM((CB,), jnp.float32),
            pltpu.VMEM((CB, CG), jnp.float32),
            pltpu.VMEM((CB, CG), jnp.float32),
            pltpu.SemaphoreType.DMA,
            pltpu.SemaphoreType.DMA,
        ],
    )
    def k(tab_hbm, gx_hbm, gy_hbm, out_hbm,
          tab_v, gxv0, gxv1, gyv0, gyv1, outv0, outv1, semg, semo):
        cid = lax.axis_index("c")
        sid = lax.axis_index("s")
        wid = sid * 2 + cid
        grp = wid // WPG
        sub = wid % WPG
        base = sub * RW

        pltpu.sync_copy(tab_hbm.at[pl.ds(grp * TS, TS)], tab_v)

        iota = lax.iota(jnp.int32, L)
        gxv = (gxv0, gxv1)
        gyv = (gyv0, gyv1)
        outv = (outv0, outv1)

        pltpu.async_copy(gx_hbm.at[pl.ds(base, CB)], gxv0, semg)
        pltpu.async_copy(gy_hbm.at[pl.ds(base, CB)], gyv0, semg)

        @pl.loop(0, NCHUNK // 2)
        def _chunk2(ii):
            for b in range(2):
                i = ii * 2 + b
                r0 = base + i * CB
                gxb, gyb, ob = gxv[b], gyv[b], outv[b]
                gxn, gyn = gxv[1 - b], gyv[1 - b]

                pltpu.make_async_copy(gx_hbm.at[pl.ds(0, CB)], gxb, semg).wait()
                pltpu.make_async_copy(gy_hbm.at[pl.ds(0, CB)], gyb, semg).wait()

                @pl.when(i + 1 < NCHUNK)
                def _pf():
                    r1 = base + (i + 1) * CB
                    pltpu.async_copy(gx_hbm.at[pl.ds(r1, CB)], gxn, semg)
                    pltpu.async_copy(gy_hbm.at[pl.ds(r1, CB)], gyn, semg)

                @pl.when(i >= 2)
                def _drain():
                    pltpu.make_async_copy(
                        ob, out_hbm.at[pl.ds(0, CB), 0], semo
                    ).wait()

                @pl.loop(0, CB // L, unroll=2)
                def _grp16(g):
                    gx16 = gxb[pl.ds(g * L, L)]
                    gy16 = gyb[pl.ds(g * L, L)]
                    px = gx16 * (W / 2.0) + (W / 2.0 - 0.5)
                    py = gy16 * (H / 2.0) + (H / 2.0 - 0.5)
                    tx = px.astype(jnp.int32).astype(jnp.float32)
                    ty = py.astype(jnp.int32).astype(jnp.float32)
                    xw = jnp.where(px < tx, tx - 1.0, tx)
                    yn = jnp.where(py < ty, ty - 1.0, ty)
                    fx = px - xw
                    fy = py - yn
                    gx1 = 1.0 - fx
                    gy1 = 1.0 - fy
                    xe = xw + 1.0
                    ys = yn + 1.0
                    wm = (xw > -1.0) & (xw < float(W))
                    em = (xe > -1.0) & (xe < float(W))
                    nm = (yn > -1.0) & (yn < float(H))
                    sm = (ys > -1.0) & (ys < float(H))
                    mnw = wm & nm
                    mne = em & nm
                    msw = wm & sm
                    mse = em & sm
                    zero = jnp.zeros((L,), jnp.float32)
                    w_nw = jnp.where(mnw, gy1 * gx1, zero)
                    w_ne = jnp.where(mne, gy1 * fx, zero)
                    w_sw = jnp.where(msw, fy * gx1, zero)
                    w_se = jnp.where(mse, fy * fx, zero)
                    bn = yn * float(W)
                    bs_ = ys * float(W)
                    i_nw = jnp.where(mnw, bn + xw, zero).astype(jnp.int32) * CG
                    i_ne = jnp.where(mne, bn + xe, zero).astype(jnp.int32) * CG
                    i_sw = jnp.where(msw, bs_ + xw, zero).astype(jnp.int32) * CG
                    i_se = jnp.where(mse, bs_ + xe, zero).astype(jnp.int32) * CG
                    rloc = g * L + iota
                    for c in range(CG):
                        v_nw = plsc.load_gather(tab_v, [i_nw + c])
                        v_ne = plsc.load_gather(tab_v, [i_ne + c])
                        v_sw = plsc.load_gather(tab_v, [i_sw + c])
                        v_se = plsc.load_gather(tab_v, [i_se + c])
                        acc = (w_nw * v_nw + w_ne * v_ne) + (
                            w_sw * v_sw + w_se * v_se
                        )
                        plsc.store_scatter(
                            ob, [rloc, jnp.full((L,), c, jnp.int32)], acc
                        )

                pltpu.async_copy(ob, out_hbm.at[pl.ds(r0, CB), grp], semo)

        pltpu.make_async_copy(outv0, out_hbm.at[pl.ds(0, CB), 0], semo).wait()
        pltpu.make_async_copy(outv1, out_hbm.at[pl.ds(0, CB), 0], semo).wait()

    return k(tab_g, gx, gy)


def kernel(source, sparse_motions):
    table = source.reshape(H * W, C)
    tab_g = table.reshape(H * W, NGRP, CG).transpose(1, 0, 2).reshape(-1)
    sm = sparse_motions.reshape(R, 2)
    gx = sm[:, 0]
    gy = sm[:, 1]
    out = _sc_deform(tab_g, gx, gy)
    return out.reshape(-1, H * W, C)


# speedup vs baseline: 2.6192x; 2.6192x over previous
import functools

import jax
import jax.numpy as jnp
from jax import lax
from jax.experimental import pallas as pl
from jax.experimental.pallas import tpu as pltpu
from jax.experimental.pallas import tpu_sc as plsc

NUM_KP = 10
H = 128
W = 128
C = 32
BS = 4

R = BS * (NUM_KP + 1) * H * W
NGRP = 8
CG = C // NGRP
WPG = 4
RW = R // WPG
CB = 1024
NCHUNK = RW // CB
L = 16
TS = H * W * CG


def _sc_deform(tab_g, gx, gy):
    mesh = plsc.VectorSubcoreMesh(
        core_axis_name="c", subcore_axis_name="s", num_cores=2, num_subcores=16
    )

    @functools.partial(
        pl.kernel,
        out_type=jax.ShapeDtypeStruct((NGRP * R * CG,), jnp.float32),
        mesh=mesh,
        compiler_params=pltpu.CompilerParams(
            needs_layout_passes=False, use_tc_tiling_on_sc=False
        ),
        scratch_types=[
            pltpu.VMEM((TS,), jnp.float32),
            pltpu.VMEM((CB,), jnp.float32),
            pltpu.VMEM((CB,), jnp.float32),
            pltpu.VMEM((CB,), jnp.float32),
            pltpu.VMEM((CB,), jnp.float32),
            pltpu.VMEM((CB * CG,), jnp.float32),
            pltpu.VMEM((CB * CG,), jnp.float32),
            pltpu.SemaphoreType.DMA,
            pltpu.SemaphoreType.DMA,
        ],
    )
    def k(tab_hbm, gx_hbm, gy_hbm, out_hbm,
          tab_v, gxv0, gxv1, gyv0, gyv1, outv0, outv1, semg, semo):
        cid = lax.axis_index("c")
        sid = lax.axis_index("s")
        wid = sid * 2 + cid
        grp = wid // WPG
        sub = wid % WPG
        base = sub * RW

        pltpu.sync_copy(tab_hbm.at[pl.ds(grp * TS, TS)], tab_v)

        iota = lax.iota(jnp.int32, L)
        gxv = (gxv0, gxv1)
        gyv = (gyv0, gyv1)
        outv = (outv0, outv1)

        pltpu.async_copy(gx_hbm.at[pl.ds(base, CB)], gxv0, semg)
        pltpu.async_copy(gy_hbm.at[pl.ds(base, CB)], gyv0, semg)

        @pl.loop(0, NCHUNK // 2)
        def _chunk2(ii):
            for b in range(2):
                i = ii * 2 + b
                r0 = base + i * CB
                gxb, gyb, ob = gxv[b], gyv[b], outv[b]
                gxn, gyn = gxv[1 - b], gyv[1 - b]

                pltpu.make_async_copy(gx_hbm.at[pl.ds(0, CB)], gxb, semg).wait()
                pltpu.make_async_copy(gy_hbm.at[pl.ds(0, CB)], gyb, semg).wait()

                @pl.when(i + 1 < NCHUNK)
                def _pf():
                    r1 = base + (i + 1) * CB
                    pltpu.async_copy(gx_hbm.at[pl.ds(r1, CB)], gxn, semg)
                    pltpu.async_copy(gy_hbm.at[pl.ds(r1, CB)], gyn, semg)

                @pl.when(i >= 2)
                def _drain():
                    pltpu.make_async_copy(
                        ob, out_hbm.at[pl.ds(0, CB * CG)], semo
                    ).wait()

                @pl.loop(0, CB // L, unroll=2)
                def _grp16(g):
                    gx16 = gxb[pl.ds(g * L, L)]
                    gy16 = gyb[pl.ds(g * L, L)]
                    px = gx16 * (W / 2.0) + (W / 2.0 - 0.5)
                    py = gy16 * (H / 2.0) + (H / 2.0 - 0.5)
                    tx = px.astype(jnp.int32).astype(jnp.float32)
                    ty = py.astype(jnp.int32).astype(jnp.float32)
                    xw = jnp.where(px < tx, tx - 1.0, tx)
                    yn = jnp.where(py < ty, ty - 1.0, ty)
                    fx = px - xw
                    fy = py - yn
                    gx1 = 1.0 - fx
                    gy1 = 1.0 - fy
                    xe = xw + 1.0
                    ys = yn + 1.0
                    wm = (xw > -1.0) & (xw < float(W))
                    em = (xe > -1.0) & (xe < float(W))
                    nm = (yn > -1.0) & (yn < float(H))
                    sm = (ys > -1.0) & (ys < float(H))
                    mnw = wm & nm
                    mne = em & nm
                    msw = wm & sm
                    mse = em & sm
                    zero = jnp.zeros((L,), jnp.float32)
                    w_nw = jnp.where(mnw, gy1 * gx1, zero)
                    w_ne = jnp.where(mne, gy1 * fx, zero)
                    w_sw = jnp.where(msw, fy * gx1, zero)
                    w_se = jnp.where(mse, fy * fx, zero)
                    bn = yn * float(W)
                    bs_ = ys * float(W)
                    i_nw = jnp.where(mnw, bn + xw, zero).astype(jnp.int32) * CG
                    i_ne = jnp.where(mne, bn + xe, zero).astype(jnp.int32) * CG
                    i_sw = jnp.where(msw, bs_ + xw, zero).astype(jnp.int32) * CG
                    i_se = jnp.where(mse, bs_ + xe, zero).astype(jnp.int32) * CG
                    rloc = (g * L + iota) * CG
                    for c in range(CG):
                        v_nw = plsc.load_gather(tab_v, [i_nw + c])
                        v_ne = plsc.load_gather(tab_v, [i_ne + c])
                        v_sw = plsc.load_gather(tab_v, [i_sw + c])
                        v_se = plsc.load_gather(tab_v, [i_se + c])
                        acc = (w_nw * v_nw + w_ne * v_ne) + (
                            w_sw * v_sw + w_se * v_se
                        )
                        plsc.store_scatter(ob, [rloc + c], acc)

                pltpu.async_copy(
                    ob, out_hbm.at[pl.ds((grp * R + r0) * CG, CB * CG)], semo
                )

        pltpu.make_async_copy(outv0, out_hbm.at[pl.ds(0, CB * CG)], semo).wait()
        pltpu.make_async_copy(outv1, out_hbm.at[pl.ds(0, CB * CG)], semo).wait()

    return k(tab_g, gx, gy)


CB2 = 512
RW2 = R // 32
NCHUNK2 = RW2 // CB2
INW = CB2 * CG


def _sc_interleave(inter):
    mesh = plsc.VectorSubcoreMesh(
        core_axis_name="c", subcore_axis_name="s", num_cores=2, num_subcores=16
    )

    @functools.partial(
        pl.kernel,
        out_type=jax.ShapeDtypeStruct((R * C,), jnp.float32),
        mesh=mesh,
        compiler_params=pltpu.CompilerParams(
            needs_layout_passes=False, use_tc_tiling_on_sc=False
        ),
        scratch_types=[
            pltpu.VMEM((NGRP * INW,), jnp.float32),
            pltpu.VMEM((NGRP * INW,), jnp.float32),
            pltpu.VMEM((CB2 * C,), jnp.float32),
            pltpu.VMEM((CB2 * C,), jnp.float32),
            pltpu.SemaphoreType.DMA,
            pltpu.SemaphoreType.DMA,
        ],
    )
    def k(in_hbm, out_hbm, inb0, inb1, outb0, outb1, semi, semo):
        cid = lax.axis_index("c")
        sid = lax.axis_index("s")
        wid = sid * 2 + cid
        base = wid * RW2

        iota = lax.iota(jnp.int32, L)
        iota4 = lax.shift_right_logical(iota, 2)
        iotam4 = lax.bitwise_and(iota, jnp.full((L,), 3, jnp.int32))
        inb = (inb0, inb1)
        outb = (outb0, outb1)

        def issue_in(i, buf):
            r0 = base + i * CB2
            for g in range(NGRP):
                pltpu.async_copy(
                    in_hbm.at[pl.ds((g * R + r0) * CG, INW)],
                    buf.at[pl.ds(g * INW, INW)],
                    semi,
                )

        issue_in(0, inb0)

        @pl.loop(0, NCHUNK2 // 2)
        def _chunk2(ii):
            for b in range(2):
                i = ii * 2 + b
                r0 = base + i * CB2
                ib, ob = inb[b], outb[b]

                for g in range(NGRP):
                    pltpu.make_async_copy(
                        in_hbm.at[pl.ds(0, INW)], ib.at[pl.ds(g * INW, INW)],
                        semi,
                    ).wait()

                @pl.when(i + 1 < NCHUNK2)
                def _pf():
                    issue_in(i + 1, inb[1 - b])

                @pl.when(i >= 2)
                def _drain():
                    pltpu.make_async_copy(
                        ob, out_hbm.at[pl.ds(0, CB2 * C)], semo
                    ).wait()

                for g in range(NGRP):
                    vbase = iota4 * C + (g * CG) + iotam4

                    @pl.loop(0, INW // L, unroll=4)
                    def _q(q):
                        v = ib[pl.ds(g * INW + q * L, L)]
                        plsc.store_scatter(ob, [vbase + q * (C * CG)], v)

                pltpu.async_copy(
                    ob, out_hbm.at[pl.ds(r0 * C, CB2 * C)], semo
                )

        pltpu.make_async_copy(outb0, out_hbm.at[pl.ds(0, CB2 * C)], semo).wait()
        pltpu.make_async_copy(outb1, out_hbm.at[pl.ds(0, CB2 * C)], semo).wait()

    return k(inter)


def kernel(source, sparse_motions):
    table = source.reshape(H * W, C)
    tab_g = table.reshape(H * W, NGRP, CG).transpose(1, 0, 2).reshape(-1)
    sm = sparse_motions.reshape(R, 2)
    gx = sm[:, 0]
    gy = sm[:, 1]
    inter = _sc_deform(tab_g, gx, gy)
    out = _sc_interleave(inter)
    return out.reshape(-1, H * W, C)


# speedup vs baseline: 2.6225x; 1.0013x over previous
import functools

import jax
import jax.numpy as jnp
from jax import lax
from jax.experimental import pallas as pl
from jax.experimental.pallas import tpu as pltpu
from jax.experimental.pallas import tpu_sc as plsc

NUM_KP = 10
H = 128
W = 128
C = 32
BS = 4

R = BS * (NUM_KP + 1) * H * W
NGRP = 8
CG = C // NGRP
WPG = 4
RW = R // WPG
CB = 1024
NCHUNK = RW // CB
L = 16
TS = H * W * CG


def _sc_deform(tab_g, gx, gy):
    mesh = plsc.VectorSubcoreMesh(
        core_axis_name="c", subcore_axis_name="s", num_cores=2, num_subcores=16
    )

    @functools.partial(
        pl.kernel,
        out_type=jax.ShapeDtypeStruct((NGRP * R * CG,), jnp.float32),
        mesh=mesh,
        compiler_params=pltpu.CompilerParams(
            needs_layout_passes=False, use_tc_tiling_on_sc=False
        ),
        scratch_types=[
            pltpu.VMEM((TS,), jnp.float32),
            pltpu.VMEM((CB,), jnp.float32),
            pltpu.VMEM((CB,), jnp.float32),
            pltpu.VMEM((CB,), jnp.float32),
            pltpu.VMEM((CB,), jnp.float32),
            pltpu.VMEM((CB * CG,), jnp.float32),
            pltpu.VMEM((CB * CG,), jnp.float32),
            pltpu.SemaphoreType.DMA,
            pltpu.SemaphoreType.DMA,
        ],
    )
    def k(tab_hbm, gx_hbm, gy_hbm, out_hbm,
          tab_v, gxv0, gxv1, gyv0, gyv1, outv0, outv1, semg, semo):
        cid = lax.axis_index("c")
        sid = lax.axis_index("s")
        wid = sid * 2 + cid
        grp = wid // WPG
        sub = wid % WPG
        base = sub * RW

        pltpu.sync_copy(tab_hbm.at[pl.ds(grp * TS, TS)], tab_v)

        iota = lax.iota(jnp.int32, L)
        gxv = (gxv0, gxv1)
        gyv = (gyv0, gyv1)
        outv = (outv0, outv1)

        pltpu.async_copy(gx_hbm.at[pl.ds(base, CB)], gxv0, semg)
        pltpu.async_copy(gy_hbm.at[pl.ds(base, CB)], gyv0, semg)

        @pl.loop(0, NCHUNK // 2)
        def _chunk2(ii):
            for b in range(2):
                i = ii * 2 + b
                r0 = base + i * CB
                gxb, gyb, ob = gxv[b], gyv[b], outv[b]
                gxn, gyn = gxv[1 - b], gyv[1 - b]

                pltpu.make_async_copy(gx_hbm.at[pl.ds(0, CB)], gxb, semg).wait()
                pltpu.make_async_copy(gy_hbm.at[pl.ds(0, CB)], gyb, semg).wait()

                @pl.when(i + 1 < NCHUNK)
                def _pf():
                    r1 = base + (i + 1) * CB
                    pltpu.async_copy(gx_hbm.at[pl.ds(r1, CB)], gxn, semg)
                    pltpu.async_copy(gy_hbm.at[pl.ds(r1, CB)], gyn, semg)

                @pl.when(i >= 2)
                def _drain():
                    pltpu.make_async_copy(
                        ob, out_hbm.at[pl.ds(0, CB * CG)], semo
                    ).wait()

                @pl.loop(0, CB // L, unroll=4)
                def _grp16(g):
                    gx16 = gxb[pl.ds(g * L, L)]
                    gy16 = gyb[pl.ds(g * L, L)]
                    px = gx16 * (W / 2.0) + (W / 2.0 - 0.5)
                    py = gy16 * (H / 2.0) + (H / 2.0 - 0.5)
                    tx = px.astype(jnp.int32).astype(jnp.float32)
                    ty = py.astype(jnp.int32).astype(jnp.float32)
                    xw = jnp.where(px < tx, tx - 1.0, tx)
                    yn = jnp.where(py < ty, ty - 1.0, ty)
                    fx = px - xw
                    fy = py - yn
                    gx1 = 1.0 - fx
                    gy1 = 1.0 - fy
                    xe = xw + 1.0
                    ys = yn + 1.0
                    wm = (xw > -1.0) & (xw < float(W))
                    em = (xe > -1.0) & (xe < float(W))
                    nm = (yn > -1.0) & (yn < float(H))
                    sm = (ys > -1.0) & (ys < float(H))
                    mnw = wm & nm
                    mne = em & nm
                    msw = wm & sm
                    mse = em & sm
                    zero = jnp.zeros((L,), jnp.float32)
                    w_nw = jnp.where(mnw, gy1 * gx1, zero)
                    w_ne = jnp.where(mne, gy1 * fx, zero)
                    w_sw = jnp.where(msw, fy * gx1, zero)
                    w_se = jnp.where(mse, fy * fx, zero)
                    bn = yn * float(W)
                    bs_ = ys * float(W)
                    i_nw = jnp.where(mnw, bn + xw, zero).astype(jnp.int32) * CG
                    i_ne = jnp.where(mne, bn + xe, zero).astype(jnp.int32) * CG
                    i_sw = jnp.where(msw, bs_ + xw, zero).astype(jnp.int32) * CG
                    i_se = jnp.where(mse, bs_ + xe, zero).astype(jnp.int32) * CG
                    rloc = (g * L + iota) * CG
                    for c in range(CG):
                        v_nw = plsc.load_gather(tab_v, [i_nw + c])
                        v_ne = plsc.load_gather(tab_v, [i_ne + c])
                        v_sw = plsc.load_gather(tab_v, [i_sw + c])
                        v_se = plsc.load_gather(tab_v, [i_se + c])
                        acc = (w_nw * v_nw + w_ne * v_ne) + (
                            w_sw * v_sw + w_se * v_se
                        )
                        plsc.store_scatter(ob, [rloc + c], acc)

                pltpu.async_copy(
                    ob, out_hbm.at[pl.ds((grp * R + r0) * CG, CB * CG)], semo
                )

        pltpu.make_async_copy(outv0, out_hbm.at[pl.ds(0, CB * CG)], semo).wait()
        pltpu.make_async_copy(outv1, out_hbm.at[pl.ds(0, CB * CG)], semo).wait()

    return k(tab_g, gx, gy)


CB2 = 512
RW2 = R // 32
NCHUNK2 = RW2 // CB2
INW = CB2 * CG


def _sc_interleave(inter):
    mesh = plsc.VectorSubcoreMesh(
        core_axis_name="c", subcore_axis_name="s", num_cores=2, num_subcores=16
    )

    @functools.partial(
        pl.kernel,
        out_type=jax.ShapeDtypeStruct((R * C,), jnp.float32),
        mesh=mesh,
        compiler_params=pltpu.CompilerParams(
            needs_layout_passes=False, use_tc_tiling_on_sc=False
        ),
        scratch_types=[
            pltpu.VMEM((NGRP * INW,), jnp.float32),
            pltpu.VMEM((NGRP * INW,), jnp.float32),
            pltpu.VMEM((CB2 * C,), jnp.float32),
            pltpu.VMEM((CB2 * C,), jnp.float32),
            pltpu.SemaphoreType.DMA,
            pltpu.SemaphoreType.DMA,
        ],
    )
    def k(in_hbm, out_hbm, inb0, inb1, outb0, outb1, semi, semo):
        cid = lax.axis_index("c")
        sid = lax.axis_index("s")
        wid = sid * 2 + cid
        base = wid * RW2

        iota = lax.iota(jnp.int32, L)
        iota4 = lax.shift_right_logical(iota, 2)
        iotam4 = lax.bitwise_and(iota, jnp.full((L,), 3, jnp.int32))
        inb = (inb0, inb1)
        outb = (outb0, outb1)

        def issue_in(i, buf):
            r0 = base + i * CB2
            for g in range(NGRP):
                pltpu.async_copy(
                    in_hbm.at[pl.ds((g * R + r0) * CG, INW)],
                    buf.at[pl.ds(g * INW, INW)],
                    semi,
                )

        issue_in(0, inb0)

        @pl.loop(0, NCHUNK2 // 2)
        def _chunk2(ii):
            for b in range(2):
                i = ii * 2 + b
                r0 = base + i * CB2
                ib, ob = inb[b], outb[b]

                for g in range(NGRP):
                    pltpu.make_async_copy(
                        in_hbm.at[pl.ds(0, INW)], ib.at[pl.ds(g * INW, INW)],
                        semi,
                    ).wait()

                @pl.when(i + 1 < NCHUNK2)
                def _pf():
                    issue_in(i + 1, inb[1 - b])

                @pl.when(i >= 2)
                def _drain():
                    pltpu.make_async_copy(
                        ob, out_hbm.at[pl.ds(0, CB2 * C)], semo
                    ).wait()

                for g in range(NGRP):
                    vbase = iota4 * C + (g * CG) + iotam4

                    @pl.loop(0, INW // L, unroll=4)
                    def _q(q):
                        v = ib[pl.ds(g * INW + q * L, L)]
                        plsc.store_scatter(ob, [vbase + q * (C * CG)], v)

                pltpu.async_copy(
                    ob, out_hbm.at[pl.ds(r0 * C, CB2 * C)], semo
                )

        pltpu.make_async_copy(outb0, out_hbm.at[pl.ds(0, CB2 * C)], semo).wait()
        pltpu.make_async_copy(outb1, out_hbm.at[pl.ds(0, CB2 * C)], semo).wait()

    return k(inter)


def kernel(source, sparse_motions):
    table = source.reshape(H * W, C)
    tab_g = table.reshape(H * W, NGRP, CG).transpose(1, 0, 2).reshape(-1)
    sm = sparse_motions.reshape(R, 2)
    gx = sm[:, 0]
    gy = sm[:, 1]
    inter = _sc_deform(tab_g, gx, gy)
    out = _sc_interleave(inter)
    return out.reshape(-1, H * W, C)


# speedup vs baseline: 3.8007x; 1.4493x over previous
import functools

import jax
import jax.numpy as jnp
from jax import lax
from jax.experimental import pallas as pl
from jax.experimental.pallas import tpu as pltpu
from jax.experimental.pallas import tpu_sc as plsc

NUM_KP = 10
H = 128
W = 128
C = 32
BS = 4

R = BS * (NUM_KP + 1) * H * W
NGRP = 4
CG = C // NGRP
PW = CG // 2
WPG = 8
RW = R // WPG
CB = 1024
NCHUNK = RW // CB
L = 16
TS = H * W * PW


def _sc_deform(tab_g, gx, gy):
    mesh = plsc.VectorSubcoreMesh(
        core_axis_name="c", subcore_axis_name="s", num_cores=2, num_subcores=16
    )

    @functools.partial(
        pl.kernel,
        out_type=jax.ShapeDtypeStruct((NGRP * R * CG,), jnp.float32),
        mesh=mesh,
        compiler_params=pltpu.CompilerParams(
            needs_layout_passes=False, use_tc_tiling_on_sc=False
        ),
        scratch_types=[
            pltpu.VMEM((TS,), jnp.int32),
            pltpu.VMEM((CB,), jnp.float32),
            pltpu.VMEM((CB,), jnp.float32),
            pltpu.VMEM((CB,), jnp.float32),
            pltpu.VMEM((CB,), jnp.float32),
            pltpu.VMEM((CB * CG,), jnp.float32),
            pltpu.VMEM((CB * CG,), jnp.float32),
            pltpu.SemaphoreType.DMA,
            pltpu.SemaphoreType.DMA,
        ],
    )
    def k(tab_hbm, gx_hbm, gy_hbm, out_hbm,
          tab_v, gxv0, gxv1, gyv0, gyv1, outv0, outv1, semg, semo):
        cid = lax.axis_index("c")
        sid = lax.axis_index("s")
        wid = sid * 2 + cid
        grp = wid // WPG
        sub = wid % WPG
        base = sub * RW

        pltpu.sync_copy(tab_hbm.at[pl.ds(grp * TS, TS)], tab_v)

        iota = lax.iota(jnp.int32, L)
        gxv = (gxv0, gxv1)
        gyv = (gyv0, gyv1)
        outv = (outv0, outv1)

        pltpu.async_copy(gx_hbm.at[pl.ds(base, CB)], gxv0, semg)
        pltpu.async_copy(gy_hbm.at[pl.ds(base, CB)], gyv0, semg)

        @pl.loop(0, NCHUNK // 2)
        def _chunk2(ii):
            for b in range(2):
                i = ii * 2 + b
                r0 = base + i * CB
                gxb, gyb, ob = gxv[b], gyv[b], outv[b]
                gxn, gyn = gxv[1 - b], gyv[1 - b]

                pltpu.make_async_copy(gx_hbm.at[pl.ds(0, CB)], gxb, semg).wait()
                pltpu.make_async_copy(gy_hbm.at[pl.ds(0, CB)], gyb, semg).wait()

                @pl.when(i + 1 < NCHUNK)
                def _pf():
                    r1 = base + (i + 1) * CB
                    pltpu.async_copy(gx_hbm.at[pl.ds(r1, CB)], gxn, semg)
                    pltpu.async_copy(gy_hbm.at[pl.ds(r1, CB)], gyn, semg)

                @pl.when(i >= 2)
                def _drain():
                    pltpu.make_async_copy(
                        ob, out_hbm.at[pl.ds(0, CB * CG)], semo
                    ).wait()

                @pl.loop(0, CB // L, unroll=4)
                def _grp16(g):
                    gx16 = gxb[pl.ds(g * L, L)]
                    gy16 = gyb[pl.ds(g * L, L)]
                    px = gx16 * (W / 2.0) + (W / 2.0 - 0.5)
                    py = gy16 * (H / 2.0) + (H / 2.0 - 0.5)
                    tx = px.astype(jnp.int32).astype(jnp.float32)
                    ty = py.astype(jnp.int32).astype(jnp.float32)
                    xw = jnp.where(px < tx, tx - 1.0, tx)
                    yn = jnp.where(py < ty, ty - 1.0, ty)
                    fx = px - xw
                    fy = py - yn
                    gx1 = 1.0 - fx
                    gy1 = 1.0 - fy
                    xe = xw + 1.0
                    ys = yn + 1.0
                    wm = (xw > -1.0) & (xw < float(W))
                    em = (xe > -1.0) & (xe < float(W))
                    nm = (yn > -1.0) & (yn < float(H))
                    sm = (ys > -1.0) & (ys < float(H))
                    mnw = wm & nm
                    mne = em & nm
                    msw = wm & sm
                    mse = em & sm
                    zero = jnp.zeros((L,), jnp.float32)
                    w_nw = jnp.where(mnw, gy1 * gx1, zero)
                    w_ne = jnp.where(mne, gy1 * fx, zero)
                    w_sw = jnp.where(msw, fy * gx1, zero)
                    w_se = jnp.where(mse, fy * fx, zero)
                    bn = yn * float(W)
                    bs_ = ys * float(W)
                    i_nw = jnp.where(mnw, bn + xw, zero).astype(jnp.int32) * PW
                    i_ne = jnp.where(mne, bn + xe, zero).astype(jnp.int32) * PW
                    i_sw = jnp.where(msw, bs_ + xw, zero).astype(jnp.int32) * PW
                    i_se = jnp.where(mse, bs_ + xe, zero).astype(jnp.int32) * PW
                    rloc = (g * L + iota) * CG
                    corners = ((w_nw, i_nw), (w_ne, i_ne), (w_sw, i_sw),
                               (w_se, i_se))
                    for p in range(PW):
                        acc0 = zero
                        acc1 = zero
                        for wgt, idx in corners:
                            v = plsc.load_gather(tab_v, [idx + p])
                            a, bb = plsc.unpack(
                                plsc.bitcast(v, jnp.bfloat16),
                                format=plsc.PackFormat.INTERLEAVED,
                            )
                            acc0 = acc0 + wgt * a
                            acc1 = acc1 + wgt * bb
                        plsc.store_scatter(ob, [rloc + 2 * p], acc0)
                        plsc.store_scatter(ob, [rloc + 2 * p + 1], acc1)

                pltpu.async_copy(
                    ob, out_hbm.at[pl.ds((grp * R + r0) * CG, CB * CG)], semo
                )

        pltpu.make_async_copy(outv0, out_hbm.at[pl.ds(0, CB * CG)], semo).wait()
        pltpu.make_async_copy(outv1, out_hbm.at[pl.ds(0, CB * CG)], semo).wait()

    return k(tab_g, gx, gy)


CB2 = 512
RW2 = R // 32
NCHUNK2 = RW2 // CB2
INW = CB2 * CG


def _sc_interleave(inter):
    mesh = plsc.VectorSubcoreMesh(
        core_axis_name="c", subcore_axis_name="s", num_cores=2, num_subcores=16
    )

    @functools.partial(
        pl.kernel,
        out_type=jax.ShapeDtypeStruct((R * C,), jnp.float32),
        mesh=mesh,
        compiler_params=pltpu.CompilerParams(
            needs_layout_passes=False, use_tc_tiling_on_sc=False
        ),
        scratch_types=[
            pltpu.VMEM((NGRP * INW,), jnp.float32),
            pltpu.VMEM((NGRP * INW,), jnp.float32),
            pltpu.VMEM((CB2 * C,), jnp.float32),
            pltpu.VMEM((CB2 * C,), jnp.float32),
            pltpu.SemaphoreType.DMA,
            pltpu.SemaphoreType.DMA,
        ],
    )
    def k(in_hbm, out_hbm, inb0, inb1, outb0, outb1, semi, semo):
        cid = lax.axis_index("c")
        sid = lax.axis_index("s")
        wid = sid * 2 + cid
        base = wid * RW2

        iota = lax.iota(jnp.int32, L)
        iotac = lax.shift_right_logical(iota, 3)
        iotamc = lax.bitwise_and(iota, jnp.full((L,), CG - 1, jnp.int32))
        inb = (inb0, inb1)
        outb = (outb0, outb1)

        def issue_in(i, buf):
            r0 = base + i * CB2
            for g in range(NGRP):
                pltpu.async_copy(
                    in_hbm.at[pl.ds((g * R + r0) * CG, INW)],
                    buf.at[pl.ds(g * INW, INW)],
                    semi,
                )

        issue_in(0, inb0)

        @pl.loop(0, NCHUNK2 // 2)
        def _chunk2(ii):
            for b in range(2):
                i = ii * 2 + b
                r0 = base + i * CB2
                ib, ob = inb[b], outb[b]

                for g in range(NGRP):
                    pltpu.make_async_copy(
                        in_hbm.at[pl.ds(0, INW)], ib.at[pl.ds(g * INW, INW)],
                        semi,
                    ).wait()

                @pl.when(i + 1 < NCHUNK2)
                def _pf():
                    issue_in(i + 1, inb[1 - b])

                @pl.when(i >= 2)
                def _drain():
                    pltpu.make_async_copy(
                        ob, out_hbm.at[pl.ds(0, CB2 * C)], semo
                    ).wait()

                for g in range(NGRP):
                    vbase = iotac * C + (g * CG) + iotamc

                    @pl.loop(0, INW // L, unroll=4)
                    def _q(q):
                        v = ib[pl.ds(g * INW + q * L, L)]
                        plsc.store_scatter(ob, [vbase + q * ((L // CG) * C)], v)

                pltpu.async_copy(
                    ob, out_hbm.at[pl.ds(r0 * C, CB2 * C)], semo
                )

        pltpu.make_async_copy(outb0, out_hbm.at[pl.ds(0, CB2 * C)], semo).wait()
        pltpu.make_async_copy(outb1, out_hbm.at[pl.ds(0, CB2 * C)], semo).wait()

    return k(inter)


def kernel(source, sparse_motions):
    table = source.reshape(H * W, C).astype(jnp.bfloat16)
    bits = jax.lax.bitcast_convert_type(table, jnp.uint16)
    lo = bits[:, 0::2].astype(jnp.uint32)
    hi = bits[:, 1::2].astype(jnp.uint32)
    packed = jax.lax.bitcast_convert_type(lo | (hi << 16), jnp.int32)
    tab_g = packed.reshape(H * W, NGRP, PW).transpose(1, 0, 2).reshape(-1)
    sm = sparse_motions.reshape(R, 2)
    gx = sm[:, 0]
    gy = sm[:, 1]
    inter = _sc_deform(tab_g, gx, gy)
    out = _sc_interleave(inter)
    return out.reshape(-1, H * W, C)
